# baseline (speedup 1.0000x reference)
import jax
import jax.numpy as jnp
from jax.experimental import pallas as pl
from jax.experimental.pallas import tpu as pltpu

BT = 2048
DSPLIT = 2
DC = 1024 // DSPLIT


def _route(logits, logits_ref, probs_ref, mask_ref):
    logits_ref[...] = logits
    m1 = jnp.max(logits, axis=0, keepdims=True)
    ex = jnp.exp(logits - m1)
    probs_ref[...] = ex / jnp.sum(ex, axis=0, keepdims=True)
    e = logits.shape[0]
    row = jax.lax.broadcasted_iota(jnp.int32, logits.shape, 0)
    cand1 = jnp.where(logits == m1, row, e)
    i1 = jnp.min(cand1, axis=0, keepdims=True)
    take1 = row == i1
    v2 = jnp.where(take1, -jnp.inf, logits)
    m2 = jnp.max(v2, axis=0, keepdims=True)
    cand2 = jnp.where(v2 == m2, row, e)
    i2 = jnp.min(cand2, axis=0, keepdims=True)
    mask_ref[...] = (take1 | (row == i2)).astype(mask_ref.dtype)


def _router_body(h_ref, w_ref, logits_ref, probs_ref, mask_ref, acc_ref):
    d = pl.program_id(1)
    part = jax.lax.dot_general(
        w_ref[...], h_ref[...], (((1,), (1,)), ((), ())),
        preferred_element_type=jnp.float32,
    )

    @pl.when(d == 0)
    def _():
        acc_ref[...] = part

    @pl.when(d == DSPLIT - 1)
    def _():
        logits = acc_ref[...] + part
        _route(logits, logits_ref, probs_ref, mask_ref)


@jax.jit
def kernel(h, W):
    t, d = h.shape
    e = W.shape[0]
    grid = (t // BT, DSPLIT)
    logits_t, probs_t, mask_t = pl.pallas_call(
        _router_body,
        grid=grid,
        in_specs=[
            pl.BlockSpec((BT, DC), lambda i, j: (i, j)),
            pl.BlockSpec((e, DC), lambda i, j: (0, j)),
        ],
        out_specs=[pl.BlockSpec((e, BT), lambda i, j: (0, i))] * 3,
        out_shape=[jax.ShapeDtypeStruct((e, t), jnp.float32)] * 3,
        scratch_shapes=[pltpu.VMEM((e, BT), jnp.float32)],
        compiler_params=pltpu.CompilerParams(
            dimension_semantics=("arbitrary", "arbitrary"),
        ),
    )(h, W)
    logits = logits_t.T
    return (mask_t.T.astype(bool), probs_t.T, logits, logits)
